# baseline (speedup 1.0000x reference)
import functools

import jax
import jax.numpy as jnp
from jax import lax
from jax.experimental import pallas as pl
from jax.experimental.pallas import tpu as pltpu
from jax.experimental.pallas import tpu_sc as plsc

NC = 2
NS = 16
LANES = 16
DEGW = 8
K = 40

f32 = jnp.float32


def _deg_body(n, nch, edges_hbm, ones_hbm, zeros_hbm, out_hbm,
              idx_s, idx_d, ones_v, deg_out_sh, deg_in_sh, sem_a, sem_b):
    c = lax.axis_index("c")
    s = lax.axis_index("s")
    rpt = n // NS
    base = s * rpt
    lag = 8
    pt = nch * K
    eoff = (c * NS + s) * pt
    pltpu.sync_copy(edges_hbm.at[0, pl.ds(eoff, pt)], idx_s)
    pltpu.sync_copy(edges_hbm.at[1, pl.ds(eoff, pt)], idx_d)
    pltpu.sync_copy(ones_hbm, ones_v)
    pltpu.sync_copy(zeros_hbm, deg_out_sh.at[pl.ds(base, rpt)])
    pltpu.sync_copy(zeros_hbm, deg_in_sh.at[pl.ds(base, rpt)])
    plsc.subcore_barrier()

    @pl.loop(0, nch)
    def _chunk(j):
        @pl.when(j >= lag)
        def _():
            pltpu.make_async_copy(ones_v, deg_out_sh.at[idx_s.at[pl.ds(j * K, K)]], sem_a).wait()
            pltpu.make_async_copy(ones_v, deg_in_sh.at[idx_d.at[pl.ds(j * K, K)]], sem_b).wait()
        pltpu.async_copy(ones_v, deg_out_sh.at[idx_s.at[pl.ds(j * K, K)]], sem_a, add=True)
        pltpu.async_copy(ones_v, deg_in_sh.at[idx_d.at[pl.ds(j * K, K)]], sem_b, add=True)

    for _ in range(lag):
        pltpu.make_async_copy(ones_v, deg_out_sh.at[idx_s.at[pl.ds(0, K)]], sem_a).wait()
        pltpu.make_async_copy(ones_v, deg_in_sh.at[idx_d.at[pl.ds(0, K)]], sem_b).wait()
    plsc.subcore_barrier()
    pltpu.sync_copy(deg_out_sh.at[pl.ds(base, rpt)],
                    out_hbm.at[c, 0, pl.ds(base, rpt)])
    pltpu.sync_copy(deg_in_sh.at[pl.ds(base, rpt)],
                    out_hbm.at[c, 1, pl.ds(base, rpt)])


def _deg_call(edges, n, nch):
    mesh = plsc.VectorSubcoreMesh(core_axis_name="c", subcore_axis_name="s")
    body = functools.partial(_deg_body, n, nch)
    ones = jnp.ones((K, DEGW), f32)
    zeros = jnp.zeros((n // NS, DEGW), f32)
    return pl.kernel(
        body,
        out_type=jax.ShapeDtypeStruct((NC, 2, n, DEGW), f32),
        mesh=mesh,
        compiler_params=pltpu.CompilerParams(use_tc_tiling_on_sc=False),
        scratch_types=[
            pltpu.VMEM((nch * K,), jnp.int32),
            pltpu.VMEM((nch * K,), jnp.int32),
            pltpu.VMEM((K, DEGW), f32),
            pltpu.VMEM_SHARED((n, DEGW), f32),
            pltpu.VMEM_SHARED((n, DEGW), f32),
            pltpu.SemaphoreType.DMA,
            pltpu.SemaphoreType.DMA,
        ],
    )(edges, ones, zeros)


def _agg_body(n, nch, nbuf, h_hbm, edges_hbm, zeros_hbm, out_hbm,
              idx_s, idx_d, agg_sh, *bufs):
    msg = bufs[:nbuf]
    gsem = bufs[nbuf:2 * nbuf]
    ssem = bufs[2 * nbuf:]
    c = lax.axis_index("c")
    s = lax.axis_index("s")
    rpt = n // NS
    base = s * rpt
    pt = nch * K
    eoff = (c * NS + s) * pt
    pltpu.sync_copy(edges_hbm.at[0, pl.ds(eoff, pt)], idx_s)
    pltpu.sync_copy(edges_hbm.at[1, pl.ds(eoff, pt)], idx_d)
    pltpu.sync_copy(zeros_hbm, agg_sh.at[pl.ds(base, rpt)])
    plsc.subcore_barrier()

    for b in range(nbuf):
        pltpu.async_copy(h_hbm.at[idx_s.at[pl.ds(b * K, K)]], msg[b], gsem[b])

    @pl.loop(0, nch, step=nbuf)
    def _outer(i):
        for b in range(nbuf):
            j = i + b

            @pl.when(j < nch)
            def _():
                pltpu.make_async_copy(h_hbm.at[idx_s.at[pl.ds(j * K, K)]], msg[b], gsem[b]).wait()
                pltpu.async_copy(msg[b], agg_sh.at[idx_d.at[pl.ds(j * K, K)]], ssem[b], add=True)
        for b in range(nbuf):
            j2 = i + b + nbuf

            @pl.when(j2 < nch)
            def _():
                pltpu.make_async_copy(msg[b], agg_sh.at[idx_d.at[pl.ds(j2 * K, K)]], ssem[b]).wait()
                pltpu.async_copy(h_hbm.at[idx_s.at[pl.ds(j2 * K, K)]], msg[b], gsem[b])

    for b in range(min(nbuf, nch)):
        pltpu.make_async_copy(msg[b], agg_sh.at[idx_d.at[pl.ds(0, K)]], ssem[b]).wait()
    plsc.subcore_barrier()
    pltpu.sync_copy(agg_sh.at[pl.ds(base, rpt)], out_hbm.at[c, pl.ds(base, rpt)])


def _agg_call(h, edges, nch):
    n, d = h.shape
    mesh = plsc.VectorSubcoreMesh(core_axis_name="c", subcore_axis_name="s")
    nbuf = 6
    body = functools.partial(_agg_body, n, nch, nbuf)
    zeros = jnp.zeros((n // NS, d), f32)
    return pl.kernel(
        body,
        out_type=jax.ShapeDtypeStruct((NC, n, d), f32),
        mesh=mesh,
        compiler_params=pltpu.CompilerParams(use_tc_tiling_on_sc=False),
        scratch_types=[
            pltpu.VMEM((nch * K,), jnp.int32),
            pltpu.VMEM((nch * K,), jnp.int32),
            pltpu.VMEM_SHARED((n, d), f32),
        ] + [pltpu.VMEM((K, d), f32)] * nbuf
          + [pltpu.SemaphoreType.DMA] * (2 * nbuf),
    )(h, edges, zeros)


def _mm_body(x_ref, w_ref, h_ref):
    h_ref[...] = jnp.dot(x_ref[...], w_ref[...], preferred_element_type=f32)


def _mm_call(x, w1):
    n, d_in = x.shape
    hid = w1.shape[1]
    br = 2000
    return pl.pallas_call(
        _mm_body,
        grid=(n // br,),
        in_specs=[
            pl.BlockSpec((br, d_in), lambda i: (i, 0)),
            pl.BlockSpec((d_in, hid), lambda i: (0, 0)),
        ],
        out_specs=pl.BlockSpec((br, hid), lambda i: (i, 0)),
        out_shape=jax.ShapeDtypeStruct((n, hid), f32),
    )(x, w1)


def _tc1_body(deg_ref, h0_ref, h_ref, sc_ref):
    dp = deg_ref[...]
    do = jnp.maximum(dp[0, 0, :, 0:1] + dp[1, 0, :, 0:1], 1.0)
    di = jnp.maximum(dp[0, 1, :, 0:1] + dp[1, 1, :, 0:1], 1.0)
    so = lax.rsqrt(do)
    si = lax.rsqrt(di)
    h_ref[...] = h0_ref[...] * so
    sc_ref[...] = jnp.concatenate([so, si], axis=1)


def _tc1_call(deg, h0):
    n, hid = h0.shape
    br = 2000
    return pl.pallas_call(
        _tc1_body,
        grid=(n // br,),
        in_specs=[
            pl.BlockSpec((NC, 2, br, DEGW), lambda i: (0, 0, i, 0)),
            pl.BlockSpec((br, hid), lambda i: (i, 0)),
        ],
        out_specs=[
            pl.BlockSpec((br, hid), lambda i: (i, 0)),
            pl.BlockSpec((br, 2), lambda i: (i, 0)),
        ],
        out_shape=[
            jax.ShapeDtypeStruct((n, hid), f32),
            jax.ShapeDtypeStruct((n, 2), f32),
        ],
    )(deg, h0)


def _tc2_body(agg_ref, sc_ref, b_ref, w_ref, h_ref):
    a = agg_ref[0] + agg_ref[1]
    sc = sc_ref[...]
    z = jnp.maximum(a * sc[:, 1:2] + b_ref[...], 0.0)
    h = jnp.dot(z, w_ref[...], preferred_element_type=f32)
    h_ref[...] = h * sc[:, 0:1]


def _tc2_call(agg, scales, b1, w2):
    _, n, hid = agg.shape
    out = w2.shape[1]
    br = 2000
    grid = n // br
    return pl.pallas_call(
        _tc2_body,
        grid=(grid,),
        in_specs=[
            pl.BlockSpec((NC, br, hid), lambda i: (0, i, 0)),
            pl.BlockSpec((br, 2), lambda i: (i, 0)),
            pl.BlockSpec((1, hid), lambda i: (0, 0)),
            pl.BlockSpec((hid, out), lambda i: (0, 0)),
        ],
        out_specs=pl.BlockSpec((br, out), lambda i: (i, 0)),
        out_shape=jax.ShapeDtypeStruct((n, out), f32),
    )(agg, scales, b1, w2)


def _tc3_body(agg_ref, sc_ref, b_ref, o_ref):
    a = (agg_ref[0] + agg_ref[1]) * sc_ref[:, 1:2] + b_ref[...]
    m = jnp.max(a, axis=1, keepdims=True)
    e = jnp.exp(a - m)
    lse = jnp.log(jnp.sum(e, axis=1, keepdims=True))
    o_ref[...] = a - m - lse


def _tc3_call(agg, scales, b2):
    _, n, out = agg.shape
    br = 2000
    grid = n // br
    return pl.pallas_call(
        _tc3_body,
        grid=(grid,),
        in_specs=[
            pl.BlockSpec((NC, br, out), lambda i: (0, i, 0)),
            pl.BlockSpec((br, 2), lambda i: (i, 0)),
            pl.BlockSpec((1, out), lambda i: (0, 0)),
        ],
        out_specs=pl.BlockSpec((br, out), lambda i: (i, 0)),
        out_shape=jax.ShapeDtypeStruct((n, out), f32),
    )(agg, scales, b2)


def kernel(features, edge_index, W1, b1, W2, b2):
    n, _ = features.shape
    e = edge_index.shape[1]
    per_tile = e // (NC * NS)
    nch = per_tile // K

    h0 = _mm_call(features, W1)
    deg = _deg_call(edge_index, n, nch)
    h1, scales = _tc1_call(deg, h0)
    agg1 = _agg_call(h1, edge_index, nch)
    h2 = _tc2_call(agg1, scales, b1.reshape(1, -1), W2)
    agg2 = _agg_call(h2, edge_index, nch)
    return _tc3_call(agg2, scales, b2.reshape(1, -1))


# speedup vs baseline: 1.0128x; 1.0128x over previous
import functools

import jax
import jax.numpy as jnp
from jax import lax
from jax.experimental import pallas as pl
from jax.experimental.pallas import tpu as pltpu
from jax.experimental.pallas import tpu_sc as plsc

NC = 2
NS = 16
LANES = 16
DEGW = 8
K = 40

f32 = jnp.float32


def _deg_body(n, nch, edges_hbm, ones_hbm, zeros_hbm, out_hbm,
              idx_s, idx_d, ones_v, deg_out_sh, deg_in_sh, sem_a, sem_b):
    c = lax.axis_index("c")
    s = lax.axis_index("s")
    rpt = n // NS
    base = s * rpt
    lag = 4
    pt = nch * K
    eoff = (c * NS + s) * pt
    pltpu.sync_copy(edges_hbm.at[0, pl.ds(eoff, pt)], idx_s)
    pltpu.sync_copy(edges_hbm.at[1, pl.ds(eoff, pt)], idx_d)
    pltpu.sync_copy(ones_hbm, ones_v)
    pltpu.sync_copy(zeros_hbm, deg_out_sh.at[pl.ds(base, rpt)])
    pltpu.sync_copy(zeros_hbm, deg_in_sh.at[pl.ds(base, rpt)])
    plsc.subcore_barrier()

    @pl.loop(0, nch)
    def _chunk(j):
        @pl.when(j >= lag)
        def _():
            pltpu.make_async_copy(ones_v, deg_out_sh.at[idx_s.at[pl.ds(j * K, K)]], sem_a).wait()
            pltpu.make_async_copy(ones_v, deg_in_sh.at[idx_d.at[pl.ds(j * K, K)]], sem_b).wait()
        pltpu.async_copy(ones_v, deg_out_sh.at[idx_s.at[pl.ds(j * K, K)]], sem_a, add=True)
        pltpu.async_copy(ones_v, deg_in_sh.at[idx_d.at[pl.ds(j * K, K)]], sem_b, add=True)

    for _ in range(lag):
        pltpu.make_async_copy(ones_v, deg_out_sh.at[idx_s.at[pl.ds(0, K)]], sem_a).wait()
        pltpu.make_async_copy(ones_v, deg_in_sh.at[idx_d.at[pl.ds(0, K)]], sem_b).wait()
    plsc.subcore_barrier()
    pltpu.sync_copy(deg_out_sh.at[pl.ds(base, rpt)],
                    out_hbm.at[c, 0, pl.ds(base, rpt)])
    pltpu.sync_copy(deg_in_sh.at[pl.ds(base, rpt)],
                    out_hbm.at[c, 1, pl.ds(base, rpt)])


def _deg_call(edges, n, nch):
    mesh = plsc.VectorSubcoreMesh(core_axis_name="c", subcore_axis_name="s")
    body = functools.partial(_deg_body, n, nch)
    ones = jnp.ones((K, DEGW), f32)
    zeros = jnp.zeros((n // NS, DEGW), f32)
    return pl.kernel(
        body,
        out_type=jax.ShapeDtypeStruct((NC, 2, n, DEGW), f32),
        mesh=mesh,
        compiler_params=pltpu.CompilerParams(use_tc_tiling_on_sc=False),
        scratch_types=[
            pltpu.VMEM((nch * K,), jnp.int32),
            pltpu.VMEM((nch * K,), jnp.int32),
            pltpu.VMEM((K, DEGW), f32),
            pltpu.VMEM_SHARED((n, DEGW), f32),
            pltpu.VMEM_SHARED((n, DEGW), f32),
            pltpu.SemaphoreType.DMA,
            pltpu.SemaphoreType.DMA,
        ],
    )(edges, ones, zeros)


def _agg_body(n, nch, nbuf, h_hbm, edges_hbm, zeros_hbm, out_hbm,
              idx_s, idx_d, agg_sh, *bufs):
    msg = bufs[:nbuf]
    gsem = bufs[nbuf:2 * nbuf]
    ssem = bufs[2 * nbuf:]
    c = lax.axis_index("c")
    s = lax.axis_index("s")
    rpt = n // NS
    base = s * rpt
    pt = nch * K
    eoff = (c * NS + s) * pt
    pltpu.sync_copy(edges_hbm.at[0, pl.ds(eoff, pt)], idx_s)
    pltpu.sync_copy(edges_hbm.at[1, pl.ds(eoff, pt)], idx_d)
    pltpu.sync_copy(zeros_hbm, agg_sh.at[pl.ds(base, rpt)])
    plsc.subcore_barrier()

    for b in range(nbuf):
        pltpu.async_copy(h_hbm.at[idx_s.at[pl.ds(b * K, K)]], msg[b], gsem[b])

    @pl.loop(0, nch, step=nbuf)
    def _outer(i):
        for b in range(nbuf):
            j = i + b

            @pl.when(j < nch)
            def _():
                pltpu.make_async_copy(h_hbm.at[idx_s.at[pl.ds(j * K, K)]], msg[b], gsem[b]).wait()
                pltpu.async_copy(msg[b], agg_sh.at[idx_d.at[pl.ds(j * K, K)]], ssem[b], add=True)
        for b in range(nbuf):
            j2 = i + b + nbuf

            @pl.when(j2 < nch)
            def _():
                pltpu.make_async_copy(msg[b], agg_sh.at[idx_d.at[pl.ds(j2 * K, K)]], ssem[b]).wait()
                pltpu.async_copy(h_hbm.at[idx_s.at[pl.ds(j2 * K, K)]], msg[b], gsem[b])

    for b in range(min(nbuf, nch)):
        pltpu.make_async_copy(msg[b], agg_sh.at[idx_d.at[pl.ds(0, K)]], ssem[b]).wait()
    plsc.subcore_barrier()
    pltpu.sync_copy(agg_sh.at[pl.ds(base, rpt)], out_hbm.at[c, pl.ds(base, rpt)])


def _agg_call(h, edges, nch):
    n, d = h.shape
    mesh = plsc.VectorSubcoreMesh(core_axis_name="c", subcore_axis_name="s")
    nbuf = 6
    body = functools.partial(_agg_body, n, nch, nbuf)
    zeros = jnp.zeros((n // NS, d), f32)
    return pl.kernel(
        body,
        out_type=jax.ShapeDtypeStruct((NC, n, d), f32),
        mesh=mesh,
        compiler_params=pltpu.CompilerParams(use_tc_tiling_on_sc=False),
        scratch_types=[
            pltpu.VMEM((nch * K,), jnp.int32),
            pltpu.VMEM((nch * K,), jnp.int32),
            pltpu.VMEM_SHARED((n, d), f32),
        ] + [pltpu.VMEM((K, d), f32)] * nbuf
          + [pltpu.SemaphoreType.DMA] * (2 * nbuf),
    )(h, edges, zeros)


def _tc1_body(deg_ref, x_ref, w_ref, h_ref, sc_ref):
    dp = deg_ref[...]
    do = jnp.maximum(dp[0, 0, :, 0:1] + dp[1, 0, :, 0:1], 1.0)
    di = jnp.maximum(dp[0, 1, :, 0:1] + dp[1, 1, :, 0:1], 1.0)
    so = lax.rsqrt(do)
    si = lax.rsqrt(di)
    h = jnp.dot(x_ref[...], w_ref[...], preferred_element_type=f32)
    h_ref[...] = h * so
    sc_ref[...] = jnp.concatenate([so, si], axis=1)


def _tc1_call(deg, x, w1):
    n, d_in = x.shape
    hid = w1.shape[1]
    br = 2000
    return pl.pallas_call(
        _tc1_body,
        grid=(n // br,),
        in_specs=[
            pl.BlockSpec((NC, 2, br, DEGW), lambda i: (0, 0, i, 0)),
            pl.BlockSpec((br, d_in), lambda i: (i, 0)),
            pl.BlockSpec((d_in, hid), lambda i: (0, 0)),
        ],
        out_specs=[
            pl.BlockSpec((br, hid), lambda i: (i, 0)),
            pl.BlockSpec((br, 2), lambda i: (i, 0)),
        ],
        out_shape=[
            jax.ShapeDtypeStruct((n, hid), f32),
            jax.ShapeDtypeStruct((n, 2), f32),
        ],
    )(deg, x, w1)


def _tc2_body(agg_ref, sc_ref, b_ref, w_ref, h_ref):
    a = agg_ref[0] + agg_ref[1]
    sc = sc_ref[...]
    z = jnp.maximum(a * sc[:, 1:2] + b_ref[...], 0.0)
    h = jnp.dot(z, w_ref[...], preferred_element_type=f32)
    h_ref[...] = h * sc[:, 0:1]


def _tc2_call(agg, scales, b1, w2):
    _, n, hid = agg.shape
    out = w2.shape[1]
    br = 2000
    grid = n // br
    return pl.pallas_call(
        _tc2_body,
        grid=(grid,),
        in_specs=[
            pl.BlockSpec((NC, br, hid), lambda i: (0, i, 0)),
            pl.BlockSpec((br, 2), lambda i: (i, 0)),
            pl.BlockSpec((1, hid), lambda i: (0, 0)),
            pl.BlockSpec((hid, out), lambda i: (0, 0)),
        ],
        out_specs=pl.BlockSpec((br, out), lambda i: (i, 0)),
        out_shape=jax.ShapeDtypeStruct((n, out), f32),
    )(agg, scales, b1, w2)


def _tc3_body(agg_ref, sc_ref, b_ref, o_ref):
    a = (agg_ref[0] + agg_ref[1]) * sc_ref[:, 1:2] + b_ref[...]
    m = jnp.max(a, axis=1, keepdims=True)
    e = jnp.exp(a - m)
    lse = jnp.log(jnp.sum(e, axis=1, keepdims=True))
    o_ref[...] = a - m - lse


def _tc3_call(agg, scales, b2):
    _, n, out = agg.shape
    br = 2000
    grid = n // br
    return pl.pallas_call(
        _tc3_body,
        grid=(grid,),
        in_specs=[
            pl.BlockSpec((NC, br, out), lambda i: (0, i, 0)),
            pl.BlockSpec((br, 2), lambda i: (i, 0)),
            pl.BlockSpec((1, out), lambda i: (0, 0)),
        ],
        out_specs=pl.BlockSpec((br, out), lambda i: (i, 0)),
        out_shape=jax.ShapeDtypeStruct((n, out), f32),
    )(agg, scales, b2)


def kernel(features, edge_index, W1, b1, W2, b2):
    n, _ = features.shape
    e = edge_index.shape[1]
    per_tile = e // (NC * NS)
    nch = per_tile // K

    deg = _deg_call(edge_index, n, nch)
    h1, scales = _tc1_call(deg, features, W1)
    agg1 = _agg_call(h1, edge_index, nch)
    h2 = _tc2_call(agg1, scales, b1.reshape(1, -1), W2)
    agg2 = _agg_call(h2, edge_index, nch)
    return _tc3_call(agg2, scales, b2.reshape(1, -1))
